# baseline (speedup 1.0000x reference)
import functools

import jax
import jax.numpy as jnp
from jax import lax
from jax.experimental import pallas as pl
from jax.experimental.pallas import tpu as pltpu
from jax.experimental.pallas import tpu_sc as plsc

D = 128
MAXLEN = 512
NSEG = 2
NC = 2
NS = 16
NW = NC * NS
CHUNK = 128
LANES = 16
SUB = 4
SUBROWS = CHUNK // SUB


def _sc_body(tok_hbm, pos_hbm, seg_emb_hbm, x_hbm, seg_hbm, out_hbm,
             psum_shr, xall, call, ident, tok0, ps0, tok1, ps1,
             semt0, semp0, semo0, semt1, semp1, semo1):
    nchunk_w = xall.shape[0]
    rows_per_w = nchunk_w * CHUNK
    w = lax.axis_index("s") * NC + lax.axis_index("c")
    iota = lax.iota(jnp.int32, LANES)
    toks = (tok0, tok1)
    pss = (ps0, ps1)
    semts = (semt0, semt1)
    semps = (semp0, semp1)
    semos = (semo0, semo1)

    sid = lax.axis_index("s")
    prows = (MAXLEN * NSEG) // NS
    g = sid // (MAXLEN // prows)
    s0 = lax.rem(sid * prows, MAXLEN)
    pltpu.sync_copy(seg_emb_hbm, tok0.at[pl.ds(0, NSEG)])
    pltpu.sync_copy(pos_hbm.at[pl.ds(s0, prows)], ps0.at[pl.ds(0, prows)])

    def prow_body(r, carry):
        for j in range(D // LANES):
            sl = pl.ds(j * LANES, LANES)
            ps0[r, sl] = ps0[r, sl] + tok0[g, sl]
        return carry

    lax.fori_loop(0, prows, prow_body, 0, unroll=2)
    pltpu.sync_copy(ps0.at[pl.ds(0, prows)],
                    psum_shr.at[pl.ds(sid * prows, prows)])
    plsc.subcore_barrier()

    pltpu.sync_copy(x_hbm.at[pl.ds(w * nchunk_w, nchunk_w)], xall)
    pltpu.sync_copy(seg_hbm.at[pl.ds(w * nchunk_w, nchunk_w)], call)

    def cidx_body(j, carry):
        posbase = lax.rem(j, MAXLEN // CHUNK) * CHUNK
        for i in range(CHUNK // LANES):
            sl = pl.ds(i * LANES, LANES)
            call[j, sl] = call[j, sl] * MAXLEN + (iota + (i * LANES + posbase))
        return carry

    lax.fori_loop(0, nchunk_w, cidx_body, 0)

    for q in range(SUB):
        for i in range(SUBROWS // LANES):
            ident[q, pl.ds(i * LANES, LANES)] = iota + (q * SUBROWS + i * LANES)

    def fire_gathers(c, b):
        cpt = pltpu.async_copy(tok_hbm.at[xall.at[c]], toks[b], semts[b])
        cpp = pltpu.async_copy(psum_shr.at[call.at[c]], pss[b], semps[b])
        return cpt, cpp

    def turn(c, b, refire):
        pltpu.make_async_copy(tok_hbm.at[xall.at[c]], toks[b], semts[b]).wait()
        pltpu.make_async_copy(psum_shr.at[call.at[c]], pss[b], semps[b]).wait()
        rowbase = (w * nchunk_w + c) * CHUNK
        out_handles = []
        for q in range(SUB):
            @plsc.parallel_loop(q * SUBROWS, (q + 1) * SUBROWS, unroll=4)
            def add_body(r):
                for j in range(D // LANES):
                    sl = pl.ds(j * LANES, LANES)
                    plsc.addupdate(pss[b].at[r, sl], toks[b][r, sl])

            out_handles.append(pltpu.async_copy(
                pss[b].at[pl.ds(q * SUBROWS, SUBROWS)],
                out_hbm.at[pl.ds(rowbase + q * SUBROWS, SUBROWS)],
                semos[b]))
        if refire:
            pltpu.async_copy(tok_hbm.at[xall.at[c + 2]], toks[b], semts[b])
        for h in out_handles:
            h.wait()
        if refire:
            pltpu.async_copy(psum_shr.at[call.at[c + 2]], pss[b], semps[b])

    fire_gathers(0, 0)
    fire_gathers(1, 1)

    def main_body(cc, carry):
        for b in range(2):
            turn(2 * cc + b, b, refire=True)
        return carry

    lax.fori_loop(0, nchunk_w // 2 - 1, main_body, 0)
    turn(nchunk_w - 2, 0, refire=False)
    turn(nchunk_w - 1, 1, refire=False)


def _sc_gather(tok_embed, pos_embed, seg_embed, x_blk, seg_blk):
    nblk = x_blk.shape[0]
    rows = nblk * CHUNK
    fn = functools.partial(
        pl.kernel,
        out_type=jax.ShapeDtypeStruct((rows, D), jnp.float32),
        mesh=plsc.VectorSubcoreMesh(core_axis_name="c", subcore_axis_name="s"),
        scratch_types=[
            pltpu.VMEM_SHARED((MAXLEN * NSEG, D), jnp.float32),
            pltpu.VMEM((nblk // NW, CHUNK), jnp.int32),
            pltpu.VMEM((nblk // NW, CHUNK), jnp.int32),
            pltpu.VMEM((SUB, SUBROWS), jnp.int32),
            pltpu.VMEM((CHUNK, D), jnp.float32),
            pltpu.VMEM((CHUNK, D), jnp.float32),
            pltpu.VMEM((CHUNK, D), jnp.float32),
            pltpu.VMEM((CHUNK, D), jnp.float32),
            pltpu.SemaphoreType.DMA,
            pltpu.SemaphoreType.DMA,
            pltpu.SemaphoreType.DMA,
            pltpu.SemaphoreType.DMA,
            pltpu.SemaphoreType.DMA,
            pltpu.SemaphoreType.DMA,
        ],
    )(_sc_body)
    return fn(tok_embed, pos_embed, seg_embed, x_blk, seg_blk)


def kernel(x, seg, tok_embed, pos_embed, seg_embed):
    batch, seqlen = x.shape
    x_blk = x.reshape(-1, CHUNK).astype(jnp.int32)
    seg_blk = seg.reshape(-1, CHUNK).astype(jnp.int32)
    out = _sc_gather(tok_embed, pos_embed, seg_embed, x_blk, seg_blk)
    return out.reshape(batch, seqlen, D)
